# baseline (speedup 1.0000x reference)
import jax
import jax.numpy as jnp
from jax import lax
from jax.experimental import pallas as pl
from jax.experimental.pallas import tpu as pltpu

_B, _S, _D = 2, 2048, 2048
_E, _R, _CH = 8, 16, 128
_NC = _S // _CH
_NCH = _B * _NC
_EP = 16
_ER = _E * _R
_H = _D // 2
_TAU = 0.7
_SCALE = 16.0 / _R
_NEG = -1e30


def _router_body(x_ref, w1_ref, b1_ref, w2_ref, b2_ref, idx_ref, cl_ref):
    i = pl.program_id(0)
    h = jnp.dot(x_ref[...], w1_ref[...], preferred_element_type=jnp.float32)
    h = jnp.maximum(h + b1_ref[...], 0.0)
    logits = jnp.dot(h, w2_ref[...], preferred_element_type=jnp.float32)
    row = jnp.mean(logits + b2_ref[...], axis=0, keepdims=True)
    cl_ref[pl.ds(i, 1), :] = row

    col16 = lax.broadcasted_iota(jnp.int32, (1, _EP), 1)
    lane32 = lax.broadcasted_iota(jnp.int32, (1, _NCH), 1)
    acc = jnp.zeros((1, _NCH), jnp.int32)

    @pl.when(i == _NCH - 1)
    def _():
        prev = jnp.zeros((1, 1), jnp.int32)
        acc2 = acc
        for c in range(_NCH):
            r = cl_ref[c:c + 1, :]
            d_new = jnp.max(r, axis=1, keepdims=True)
            am = jnp.min(jnp.where(r == d_new, col16, _EP),
                         axis=1, keepdims=True)
            d_old = jnp.sum(jnp.where(col16 == prev, r, 0.0),
                            axis=1, keepdims=True)
            take_new = ((d_new - d_old) > _TAU) if c % _NC else (d_new == d_new)
            e_c = jnp.where(take_new, am, prev)
            acc2 = jnp.where(lane32 == c, e_c, acc2)
            prev = e_c
        idx_ref[...] = acc2[None]

    @pl.when(i != _NCH - 1)
    def _():
        idx_ref[...] = acc[None]


def _route(x2d, Wr1, br1, Wr2p, br2p):
    return pl.pallas_call(
        _router_body,
        grid=(_NCH,),
        in_specs=[
            pl.BlockSpec((_CH, _D), lambda i: (i, 0)),
            pl.BlockSpec((_D, _H), lambda i: (0, 0)),
            pl.BlockSpec((1, _H), lambda i: (0, 0)),
            pl.BlockSpec((_H, _EP), lambda i: (0, 0)),
            pl.BlockSpec((1, _EP), lambda i: (0, 0)),
        ],
        out_specs=pl.BlockSpec((1, 1, _NCH), lambda i: (i, 0, 0)),
        out_shape=jax.ShapeDtypeStruct((_NCH, 1, _NCH), jnp.int32),
        scratch_shapes=[pltpu.VMEM((_NCH, _EP), jnp.float32)],
    )(x2d, Wr1, br1, Wr2p, br2p)


def _main_body(idx_sref, x_ref, wt_ref, b_ref, a_ref, bm_ref,
               out_ref, rw_ref):
    i = pl.program_id(0)
    e = idx_sref[i]
    xb = x_ref[...].astype(jnp.bfloat16)
    base = jnp.dot(xb, wt_ref[...], preferred_element_type=jnp.float32)
    ax_all = jnp.dot(xb, a_ref[...],
                     preferred_element_type=jnp.float32)
    colER = lax.broadcasted_iota(jnp.int32, (_CH, _ER), 1)
    ax = jnp.where(colER // _R == e, ax_all, 0.0)
    routed = jnp.dot(ax.astype(jnp.bfloat16), bm_ref[...],
                     preferred_element_type=jnp.float32)
    out_ref[...] = base + routed * _SCALE + b_ref[0:1, :]
    colE = lax.broadcasted_iota(jnp.int32, (_CH, _E), 1)
    rw_ref[...] = (colE == e).astype(jnp.float32)


def _main(idx_flat, x2d, Wt_bf, b8, Aall_bf, BmF_bf):
    grid_spec = pltpu.PrefetchScalarGridSpec(
        num_scalar_prefetch=1,
        grid=(_NCH,),
        in_specs=[
            pl.BlockSpec((_CH, _D), lambda i, s: (i, 0)),
            pl.BlockSpec((_D, _D), lambda i, s: (0, 0)),
            pl.BlockSpec((8, _D), lambda i, s: (0, 0)),
            pl.BlockSpec((_D, _ER), lambda i, s: (0, 0)),
            pl.BlockSpec((_ER, _D), lambda i, s: (0, 0)),
        ],
        out_specs=[
            pl.BlockSpec((_CH, _D), lambda i, s: (i, 0)),
            pl.BlockSpec((_CH, _E), lambda i, s: (i, 0)),
        ],
    )
    return pl.pallas_call(
        _main_body,
        grid_spec=grid_spec,
        out_shape=[
            jax.ShapeDtypeStruct((_B * _S, _D), jnp.float32),
            jax.ShapeDtypeStruct((_B * _S, _E), jnp.float32),
        ],
    )(idx_flat, x2d, Wt_bf, b8, Aall_bf, BmF_bf)


def kernel(x, W_base, b_base, Wr1, br1, Wr2, br2, A, Bm):
    x2d = x.reshape(_B * _S, _D)
    Wr2p = jnp.pad(Wr2, ((0, 0), (0, _EP - _E)))
    br2p = jnp.concatenate(
        [br2, jnp.full((_EP - _E,), _NEG, jnp.float32)]).reshape(1, _EP)
    br1_2d = br1.reshape(1, _H)

    idx_flat = _route(x2d, Wr1, br1_2d, Wr2p, br2p)[_NCH - 1, 0]

    Wt_bf = W_base.T.astype(jnp.bfloat16)
    Aall_bf = jnp.swapaxes(A, 0, 1).reshape(_D, _ER).astype(jnp.bfloat16)
    BmF_bf = Bm.reshape(_ER, _D).astype(jnp.bfloat16)
    b8 = jnp.broadcast_to(b_base.reshape(1, _D), (8, _D))

    out2d, rw2d = _main(idx_flat, x2d, Wt_bf, b8, Aall_bf, BmF_bf)

    output = out2d.reshape(_B, _S, _D)
    routing_weights = rw2d.reshape(_B, _S, _E)
    expert_idx = idx_flat.reshape(_B, _NC)
    return output, routing_weights, expert_idx


# speedup vs baseline: 1.0219x; 1.0219x over previous
import jax
import jax.numpy as jnp
from jax import lax
from jax.experimental import pallas as pl
from jax.experimental.pallas import tpu as pltpu

_B, _S, _D = 2, 2048, 2048
_E, _R, _CH = 8, 16, 128
_NC = _S // _CH
_NCH = _B * _NC
_EP = 16
_ER = _E * _R
_H = _D // 2
_TAU = 0.7
_SCALE = 16.0 / _R
_NEG = -1e30


def _router_body(x_ref, w1_ref, b1_ref, w2_ref, b2_ref, idx_ref, cl_ref):
    i = pl.program_id(0)
    h = jnp.dot(x_ref[...], w1_ref[...], preferred_element_type=jnp.float32)
    h = jnp.maximum(h + b1_ref[...], 0.0)
    logits = jnp.dot(h, w2_ref[...], preferred_element_type=jnp.float32)
    row = jnp.mean(logits + b2_ref[...], axis=0, keepdims=True)
    cl_ref[pl.ds(i, 1), :] = row

    lane32 = lax.broadcasted_iota(jnp.int32, (1, _NCH), 1)
    zero_row = jnp.zeros((1, _NCH), jnp.int32)

    @pl.when(i == _NCH - 1)
    def _():
        clT = cl_ref[...].T
        riota = lax.broadcasted_iota(jnp.int32, (_EP, _NCH), 0)
        d_new = jnp.max(clT, axis=0, keepdims=True)
        am = jnp.min(jnp.where(clT == d_new, riota, _EP),
                     axis=0, keepdims=True)
        am_b = jnp.broadcast_to(am, (_EP, _NCH))
        switch = (d_new - clT) > _TAU
        first = (lane32 % _NC) == 0
        T = jnp.where(switch | first, am_b, riota)
        riota1 = lax.broadcasted_iota(jnp.int32, (_EP, 1), 0)
        e = jnp.zeros((1, 1), jnp.int32)
        acc = zero_row
        for c in range(_NCH):
            e = jnp.sum(jnp.where(riota1 == e, T[:, c:c + 1], 0),
                        axis=0, keepdims=True)
            acc = jnp.where(lane32 == c, e, acc)
        idx_ref[...] = acc[None]

    @pl.when(i != _NCH - 1)
    def _():
        idx_ref[...] = zero_row[None]


def _route(x2d, Wr1, br1, Wr2p, br2p):
    return pl.pallas_call(
        _router_body,
        grid=(_NCH,),
        in_specs=[
            pl.BlockSpec((_CH, _D), lambda i: (i, 0)),
            pl.BlockSpec((_D, _H), lambda i: (0, 0)),
            pl.BlockSpec((1, _H), lambda i: (0, 0)),
            pl.BlockSpec((_H, _EP), lambda i: (0, 0)),
            pl.BlockSpec((1, _EP), lambda i: (0, 0)),
        ],
        out_specs=pl.BlockSpec((1, 1, _NCH), lambda i: (i, 0, 0)),
        out_shape=jax.ShapeDtypeStruct((_NCH, 1, _NCH), jnp.int32),
        scratch_shapes=[pltpu.VMEM((_NCH, _EP), jnp.float32)],
    )(x2d, Wr1, br1, Wr2p, br2p)


def _main_body(idx_sref, x_ref, wt_ref, b_ref, a_ref, bm_ref,
               out_ref, rw_ref):
    i = pl.program_id(0)
    e = idx_sref[i]
    xb = x_ref[...].astype(jnp.bfloat16)
    base = jnp.dot(xb, wt_ref[...], preferred_element_type=jnp.float32)
    ax_all = jnp.dot(xb, a_ref[...],
                     preferred_element_type=jnp.float32)
    colER = lax.broadcasted_iota(jnp.int32, (_CH, _ER), 1)
    ax = jnp.where(colER // _R == e, ax_all, 0.0)
    routed = jnp.dot(ax.astype(jnp.bfloat16), bm_ref[...],
                     preferred_element_type=jnp.float32)
    out_ref[...] = base + routed * _SCALE + b_ref[0:1, :]
    colE = lax.broadcasted_iota(jnp.int32, (_CH, _E), 1)
    rw_ref[...] = (colE == e).astype(jnp.float32)


def _main(idx_flat, x2d, Wt_bf, b8, Aall_bf, BmF_bf):
    grid_spec = pltpu.PrefetchScalarGridSpec(
        num_scalar_prefetch=1,
        grid=(_NCH,),
        in_specs=[
            pl.BlockSpec((_CH, _D), lambda i, s: (i, 0)),
            pl.BlockSpec((_D, _D), lambda i, s: (0, 0)),
            pl.BlockSpec((8, _D), lambda i, s: (0, 0)),
            pl.BlockSpec((_D, _ER), lambda i, s: (0, 0)),
            pl.BlockSpec((_ER, _D), lambda i, s: (0, 0)),
        ],
        out_specs=[
            pl.BlockSpec((_CH, _D), lambda i, s: (i, 0)),
            pl.BlockSpec((_CH, _E), lambda i, s: (i, 0)),
        ],
    )
    return pl.pallas_call(
        _main_body,
        grid_spec=grid_spec,
        out_shape=[
            jax.ShapeDtypeStruct((_B * _S, _D), jnp.float32),
            jax.ShapeDtypeStruct((_B * _S, _E), jnp.float32),
        ],
    )(idx_flat, x2d, Wt_bf, b8, Aall_bf, BmF_bf)


def kernel(x, W_base, b_base, Wr1, br1, Wr2, br2, A, Bm):
    x2d = x.reshape(_B * _S, _D)
    Wr2p = jnp.pad(Wr2, ((0, 0), (0, _EP - _E)))
    br2p = jnp.concatenate(
        [br2, jnp.full((_EP - _E,), _NEG, jnp.float32)]).reshape(1, _EP)
    br1_2d = br1.reshape(1, _H)

    idx_flat = _route(x2d, Wr1, br1_2d, Wr2p, br2p)[_NCH - 1, 0]

    Wt_bf = W_base.T.astype(jnp.bfloat16)
    Aall_bf = jnp.swapaxes(A, 0, 1).reshape(_D, _ER).astype(jnp.bfloat16)
    BmF_bf = Bm.reshape(_ER, _D).astype(jnp.bfloat16)
    b8 = jnp.broadcast_to(b_base.reshape(1, _D), (8, _D))

    out2d, rw2d = _main(idx_flat, x2d, Wt_bf, b8, Aall_bf, BmF_bf)

    output = out2d.reshape(_B, _S, _D)
    routing_weights = rw2d.reshape(_B, _S, _E)
    expert_idx = idx_flat.reshape(_B, _NC)
    return output, routing_weights, expert_idx
